# baseline (speedup 1.0000x reference)
import functools

import jax
import jax.numpy as jnp
from jax import lax
from jax.experimental import pallas as pl
from jax.experimental.pallas import tpu as pltpu
from jax.experimental.pallas import tpu_sc as plsc

SEQ_LEN = 200
BATCH = 4096
EMB = 50
VOCAB = 1000000
PAD_N = 16
NCLS = 2
CHUNK = 1024
NCHUNK_P = pl.cdiv(VOCAB, CHUNK)
IDX_CHUNK = 100
N_CHUNK = SEQ_LEN // IDX_CHUNK


def _sc_project(tt, w_pad):
    info = plsc.get_sparse_core_info()
    nc, ns = info.num_cores, info.num_subcores
    nw = nc * ns
    iters = pl.cdiv(NCHUNK_P, nw)
    mesh = plsc.VectorSubcoreMesh(core_axis_name="c", subcore_axis_name="s")

    @functools.partial(
        pl.kernel,
        mesh=mesh,
        compiler_params=pltpu.CompilerParams(
            use_tc_tiling_on_sc=False, needs_layout_passes=False
        ),
        out_type=jax.ShapeDtypeStruct((VOCAB, PAD_N), jnp.float32),
        scratch_types=[
            pltpu.VMEM((EMB, CHUNK), jnp.float32),
            pltpu.VMEM((CHUNK, PAD_N), jnp.float32),
            pltpu.VMEM((EMB, PAD_N), jnp.float32),
        ],
    )
    def k(tt_hbm, w_hbm, p_hbm, buf, outv, wv):
        wid = lax.axis_index("s") * nc + lax.axis_index("c")
        pltpu.sync_copy(w_hbm, wv)
        zero16 = jnp.zeros((PAD_N,), jnp.float32)
        lane = lax.iota(jnp.int32, PAD_N)

        def zbody(r, carry):
            outv[r, :] = zero16
            return carry

        lax.fori_loop(0, CHUNK, zbody, 0)

        def chunk_body(t, carry):
            cid = wid + t * nw

            @pl.when(cid < NCHUNK_P)
            def _():
                start = jnp.minimum(cid * CHUNK, VOCAB - CHUNK)
                pltpu.sync_copy(tt_hbm.at[:, pl.ds(start, CHUNK)], buf)

                def vb_body(vb, carry2):
                    base = vb * 128
                    acc0 = [jnp.zeros((16,), jnp.float32) for _ in range(8)]
                    acc1 = [jnp.zeros((16,), jnp.float32) for _ in range(8)]
                    for kk in range(EMB):
                        wrow = wv[kk, :]
                        w0 = wrow[0]
                        w1 = wrow[1]
                        for u in range(8):
                            x = buf[kk, pl.ds(base + u * 16, 16)]
                            acc0[u] = acc0[u] + x * w0
                            acc1[u] = acc1[u] + x * w1
                    for u in range(8):
                        rows = base + u * 16 + lane
                        plsc.store_scatter(outv, [rows, lane * 0], acc0[u])
                        plsc.store_scatter(outv, [rows, lane * 0 + 1], acc1[u])
                    return carry2

                lax.fori_loop(0, CHUNK // 128, vb_body, 0)
                pltpu.sync_copy(outv, p_hbm.at[pl.ds(start, CHUNK)])

            return carry

        lax.fori_loop(0, iters, chunk_body, 0)

    return k(tt, w_pad)


def _sc_lookup(xt, p, b_pad):
    info = plsc.get_sparse_core_info()
    nc, ns = info.num_cores, info.num_subcores
    nw = nc * ns
    b_per_w = BATCH // nw
    mesh = plsc.VectorSubcoreMesh(core_axis_name="c", subcore_axis_name="s")

    @functools.partial(
        pl.kernel,
        mesh=mesh,
        compiler_params=pltpu.CompilerParams(use_tc_tiling_on_sc=False),
        out_type=jax.ShapeDtypeStruct((BATCH, PAD_N), jnp.float32),
        scratch_types=[
            pltpu.VMEM((b_per_w, N_CHUNK, IDX_CHUNK), jnp.int32),
            pltpu.VMEM((2, SEQ_LEN, PAD_N), jnp.float32),
            pltpu.VMEM((b_per_w, PAD_N), jnp.float32),
            pltpu.VMEM((PAD_N,), jnp.float32),
            pltpu.SemaphoreType.DMA,
            pltpu.SemaphoreType.DMA,
        ],
    )
    def k(xt_hbm, p_hbm, b_hbm, out_hbm, idx_v, rows_v, out_v, b_v, sem0, sem1):
        wid = lax.axis_index("s") * nc + lax.axis_index("c")
        base = wid * b_per_w
        pltpu.sync_copy(xt_hbm.at[pl.ds(base, b_per_w)], idx_v)
        pltpu.sync_copy(b_hbm, b_v)
        bias = b_v[...]
        sems = (sem0, sem1)

        def gather(bi, slot, start):
            for j in range(N_CHUNK):
                cp = pltpu.make_async_copy(
                    p_hbm.at[idx_v.at[bi, j]],
                    rows_v.at[slot, pl.ds(j * IDX_CHUNK, IDX_CHUNK)],
                    sems[slot],
                )
                if start:
                    cp.start()
                else:
                    cp.wait()

        def accum(bi, slot):
            def body_s(si, acc):
                return acc + rows_v[slot, si, :]

            acc = lax.fori_loop(0, SEQ_LEN, body_s, bias, unroll=8)
            out_v[bi, :] = acc

        gather(0, 0, True)

        def body_b(i, carry):
            i0 = i * 2
            gather(i0 + 1, 1, True)
            gather(i0, 0, False)
            accum(i0, 0)

            @pl.when(i0 + 2 < b_per_w)
            def _():
                gather(i0 + 2, 0, True)

            gather(i0 + 1, 1, False)
            accum(i0 + 1, 1)
            return carry

        lax.fori_loop(0, b_per_w // 2, body_b, 0)
        pltpu.sync_copy(out_v, out_hbm.at[pl.ds(base, b_per_w)])

    return k(xt, p, b_pad)


def kernel(X, table, W, b):
    w_pad = jnp.zeros((EMB, PAD_N), jnp.float32).at[:, : W.shape[1]].set(
        W * (1.0 / SEQ_LEN)
    )
    b_pad = jnp.zeros((PAD_N,), jnp.float32).at[: b.shape[0]].set(b)
    p = _sc_project(jnp.transpose(table), w_pad)
    xt = jnp.transpose(X.astype(jnp.int32)).reshape(BATCH, N_CHUNK, IDX_CHUNK)
    out = _sc_lookup(xt, p, b_pad)
    return out[:, : W.shape[1]]


# speedup vs baseline: 1.0146x; 1.0146x over previous
import functools

import jax
import jax.numpy as jnp
from jax import lax
from jax.experimental import pallas as pl
from jax.experimental.pallas import tpu as pltpu
from jax.experimental.pallas import tpu_sc as plsc

SEQ_LEN = 200
BATCH = 4096
EMB = 50
VOCAB = 1000000
PAD_N = 16
NCLS = 2
CHUNK = 1024
NCHUNK_P = pl.cdiv(VOCAB, CHUNK)
IDX_CHUNK = 100
N_CHUNK = SEQ_LEN // IDX_CHUNK


def _sc_project(tt, w_pad):
    info = plsc.get_sparse_core_info()
    nc, ns = info.num_cores, info.num_subcores
    nw = nc * ns
    iters = pl.cdiv(NCHUNK_P, nw)
    mesh = plsc.VectorSubcoreMesh(core_axis_name="c", subcore_axis_name="s")

    @functools.partial(
        pl.kernel,
        mesh=mesh,
        compiler_params=pltpu.CompilerParams(
            use_tc_tiling_on_sc=False, needs_layout_passes=False
        ),
        out_type=jax.ShapeDtypeStruct((VOCAB, PAD_N), jnp.float32),
        scratch_types=[
            pltpu.VMEM((2, EMB, CHUNK), jnp.float32),
            pltpu.VMEM((CHUNK, PAD_N), jnp.float32),
            pltpu.VMEM((EMB, PAD_N), jnp.float32),
            pltpu.VMEM((EMB, 2, 16), jnp.float32),
            pltpu.SemaphoreType.DMA,
            pltpu.SemaphoreType.DMA,
        ],
    )
    def k(tt_hbm, w_hbm, p_hbm, buf, outv, wv, wb, sem0, sem1):
        wid = lax.axis_index("s") * nc + lax.axis_index("c")
        pltpu.sync_copy(w_hbm, wv)
        zero16 = jnp.zeros((PAD_N,), jnp.float32)
        lane = lax.iota(jnp.int32, PAD_N)
        sems = (sem0, sem1)

        for kk in range(EMB):
            wrow = wv[kk, :]
            wb[kk, 0, :] = jnp.zeros((16,), jnp.float32) + wrow[0]
            wb[kk, 1, :] = jnp.zeros((16,), jnp.float32) + wrow[1]

        def zbody(r, carry):
            outv[r, :] = zero16
            return carry

        lax.fori_loop(0, CHUNK, zbody, 0)

        def chunk_start(cid):
            return jnp.minimum(cid * CHUNK, VOCAB - CHUNK)

        def stream_in(cid, slot, start):
            s0 = chunk_start(cid)
            for kk in range(EMB):
                cp = pltpu.make_async_copy(
                    tt_hbm.at[kk, pl.ds(s0, CHUNK)],
                    buf.at[slot, kk],
                    sems[slot],
                )
                if start:
                    cp.start()
                else:
                    cp.wait()

        def compute(cid, slot):
            def vb_body(vb, carry2):
                base = vb * 128
                acc0 = [jnp.zeros((16,), jnp.float32) for _ in range(8)]
                acc1 = [jnp.zeros((16,), jnp.float32) for _ in range(8)]
                for kk in range(EMB):
                    w0 = wb[kk, 0, :]
                    w1 = wb[kk, 1, :]
                    for u in range(8):
                        x = buf[slot, kk, pl.ds(base + u * 16, 16)]
                        acc0[u] = acc0[u] + x * w0
                        acc1[u] = acc1[u] + x * w1
                for u in range(8):
                    rows = base + u * 16 + lane
                    plsc.store_scatter(outv, [rows, lane * 0], acc0[u])
                    plsc.store_scatter(outv, [rows, lane * 0 + 1], acc1[u])
                return carry2

            lax.fori_loop(0, CHUNK // 128, vb_body, 0)
            pltpu.sync_copy(outv, p_hbm.at[pl.ds(chunk_start(cid), CHUNK)])

        def step(cid, slot):
            nxt = cid + nw

            @pl.when(nxt < NCHUNK_P)
            def _():
                stream_in(nxt, 1 - slot, True)

            stream_in(cid, slot, False)
            compute(cid, slot)

        @pl.when(wid < NCHUNK_P)
        def _():
            stream_in(wid, 0, True)

        def pair_body(i, carry):
            c0 = wid + (2 * i) * nw

            @pl.when(c0 < NCHUNK_P)
            def _():
                step(c0, 0)

            c1 = wid + (2 * i + 1) * nw

            @pl.when(c1 < NCHUNK_P)
            def _():
                step(c1, 1)

            return carry

        lax.fori_loop(0, (iters + 1) // 2, pair_body, 0)

    return k(tt, w_pad)


def _sc_lookup(xt, p, b_pad):
    info = plsc.get_sparse_core_info()
    nc, ns = info.num_cores, info.num_subcores
    nw = nc * ns
    b_per_w = BATCH // nw
    mesh = plsc.VectorSubcoreMesh(core_axis_name="c", subcore_axis_name="s")

    @functools.partial(
        pl.kernel,
        mesh=mesh,
        compiler_params=pltpu.CompilerParams(use_tc_tiling_on_sc=False),
        out_type=jax.ShapeDtypeStruct((BATCH, PAD_N), jnp.float32),
        scratch_types=[
            pltpu.VMEM((b_per_w, N_CHUNK, IDX_CHUNK), jnp.int32),
            pltpu.VMEM((2, SEQ_LEN, PAD_N), jnp.float32),
            pltpu.VMEM((b_per_w, PAD_N), jnp.float32),
            pltpu.VMEM((PAD_N,), jnp.float32),
            pltpu.SemaphoreType.DMA,
            pltpu.SemaphoreType.DMA,
        ],
    )
    def k(xt_hbm, p_hbm, b_hbm, out_hbm, idx_v, rows_v, out_v, b_v, sem0, sem1):
        wid = lax.axis_index("s") * nc + lax.axis_index("c")
        base = wid * b_per_w
        pltpu.sync_copy(xt_hbm.at[pl.ds(base, b_per_w)], idx_v)
        pltpu.sync_copy(b_hbm, b_v)
        bias = b_v[...]
        sems = (sem0, sem1)

        def gather(bi, slot, start):
            for j in range(N_CHUNK):
                cp = pltpu.make_async_copy(
                    p_hbm.at[idx_v.at[bi, j]],
                    rows_v.at[slot, pl.ds(j * IDX_CHUNK, IDX_CHUNK)],
                    sems[slot],
                )
                if start:
                    cp.start()
                else:
                    cp.wait()

        def accum(bi, slot):
            def body_s(si, acc):
                return acc + rows_v[slot, si, :]

            acc = lax.fori_loop(0, SEQ_LEN, body_s, bias, unroll=8)
            out_v[bi, :] = acc

        gather(0, 0, True)

        def body_b(i, carry):
            i0 = i * 2
            gather(i0 + 1, 1, True)
            gather(i0, 0, False)
            accum(i0, 0)

            @pl.when(i0 + 2 < b_per_w)
            def _():
                gather(i0 + 2, 0, True)

            gather(i0 + 1, 1, False)
            accum(i0 + 1, 1)
            return carry

        lax.fori_loop(0, b_per_w // 2, body_b, 0)
        pltpu.sync_copy(out_v, out_hbm.at[pl.ds(base, b_per_w)])

    return k(xt, p, b_pad)


def kernel(X, table, W, b):
    w_pad = jnp.zeros((EMB, PAD_N), jnp.float32).at[:, : W.shape[1]].set(
        W * (1.0 / SEQ_LEN)
    )
    b_pad = jnp.zeros((PAD_N,), jnp.float32).at[: b.shape[0]].set(b)
    p = _sc_project(jnp.transpose(table), w_pad)
    xt = jnp.transpose(X.astype(jnp.int32)).reshape(BATCH, N_CHUNK, IDX_CHUNK)
    out = _sc_lookup(xt, p, b_pad)
    return out[:, : W.shape[1]]


# speedup vs baseline: 6.4279x; 6.3356x over previous
import functools

import jax
import jax.numpy as jnp
from jax import lax
from jax.experimental import pallas as pl
from jax.experimental.pallas import tpu as pltpu
from jax.experimental.pallas import tpu_sc as plsc

SEQ_LEN = 200
BATCH = 4096
EMB = 50
VOCAB = 1000000
PAD_N = 16
COL_BLOCK = 32768
IDX_CHUNK = 100
N_CHUNK = SEQ_LEN // IDX_CHUNK


def _proj_body(tt_ref, w_ref, p_ref):
    w = w_ref[...] * (1.0 / SEQ_LEN)
    p_ref[...] = lax.dot_general(
        tt_ref[...], w, (((0,), (0,)), ((), ())),
        preferred_element_type=jnp.float32,
    )


def _project_table(table_t, w_pad):
    grid = pl.cdiv(VOCAB, COL_BLOCK)
    return pl.pallas_call(
        _proj_body,
        grid=(grid,),
        in_specs=[
            pl.BlockSpec((EMB, COL_BLOCK), lambda i: (0, i)),
            pl.BlockSpec((EMB, PAD_N), lambda i: (0, 0)),
        ],
        out_specs=pl.BlockSpec((COL_BLOCK, PAD_N), lambda i: (i, 0)),
        out_shape=jax.ShapeDtypeStruct((VOCAB, PAD_N), jnp.float32),
    )(table_t, w_pad)


def _sc_lookup(xt, p, b_pad):
    info = plsc.get_sparse_core_info()
    nc, ns = info.num_cores, info.num_subcores
    nw = nc * ns
    b_per_w = BATCH // nw
    mesh = plsc.VectorSubcoreMesh(core_axis_name="c", subcore_axis_name="s")

    @functools.partial(
        pl.kernel,
        mesh=mesh,
        compiler_params=pltpu.CompilerParams(use_tc_tiling_on_sc=False),
        out_type=jax.ShapeDtypeStruct((BATCH, PAD_N), jnp.float32),
        scratch_types=[
            pltpu.VMEM((b_per_w, N_CHUNK, IDX_CHUNK), jnp.int32),
            pltpu.VMEM((2, SEQ_LEN, PAD_N), jnp.float32),
            pltpu.VMEM((b_per_w, PAD_N), jnp.float32),
            pltpu.VMEM((PAD_N,), jnp.float32),
            pltpu.SemaphoreType.DMA,
            pltpu.SemaphoreType.DMA,
        ],
    )
    def k(xt_hbm, p_hbm, b_hbm, out_hbm, idx_v, rows_v, out_v, b_v, sem0, sem1):
        wid = lax.axis_index("s") * nc + lax.axis_index("c")
        base = wid * b_per_w
        pltpu.sync_copy(xt_hbm.at[pl.ds(base, b_per_w)], idx_v)
        pltpu.sync_copy(b_hbm, b_v)
        bias = b_v[...]
        sems = (sem0, sem1)

        def gather(bi, slot, start):
            for j in range(N_CHUNK):
                cp = pltpu.make_async_copy(
                    p_hbm.at[idx_v.at[bi, j]],
                    rows_v.at[slot, pl.ds(j * IDX_CHUNK, IDX_CHUNK)],
                    sems[slot],
                )
                if start:
                    cp.start()
                else:
                    cp.wait()

        def accum(bi, slot):
            def body_s(si, acc):
                return acc + rows_v[slot, si, :]

            acc = lax.fori_loop(0, SEQ_LEN, body_s, bias, unroll=8)
            out_v[bi, :] = acc

        gather(0, 0, True)

        def body_b(i, carry):
            i0 = i * 2
            gather(i0 + 1, 1, True)
            gather(i0, 0, False)
            accum(i0, 0)

            @pl.when(i0 + 2 < b_per_w)
            def _():
                gather(i0 + 2, 0, True)

            gather(i0 + 1, 1, False)
            accum(i0 + 1, 1)
            return carry

        lax.fori_loop(0, b_per_w // 2, body_b, 0)
        pltpu.sync_copy(out_v, out_hbm.at[pl.ds(base, b_per_w)])

    return k(xt, p, b_pad)


def kernel(X, table, W, b):
    w_pad = jnp.zeros((EMB, PAD_N), jnp.float32).at[:, : W.shape[1]].set(W)
    b_pad = jnp.zeros((PAD_N,), jnp.float32).at[: b.shape[0]].set(b)
    p = _project_table(jnp.transpose(table), w_pad)
    xt = jnp.transpose(X.astype(jnp.int32)).reshape(BATCH, N_CHUNK, IDX_CHUNK)
    out = _sc_lookup(xt, p, b_pad)
    return out[:, : W.shape[1]]
